# baseline (speedup 1.0000x reference)
import jax
import jax.numpy as jnp
from jax import lax
from jax.experimental import pallas as pl
from jax.experimental.pallas import tpu as pltpu
from jax.experimental.pallas import tpu_sc as plsc

NUM_NODES = 1000000
DIM = 64
BATCH = 16384

NC = 2
NS = 16
L = 16
NW = NC * NS
B_PER_W = BATCH // NW
GATHER_CHUNK = 128
N_CHUNKS = B_PER_W // GATHER_CHUNK
GROUPS = B_PER_W // L


def _sc_body(src_hbm, dst_hbm, in_hbm, out_hbm, o_hbm,
             sidx_v, didx_v, a_v, b_v, s_v, o_v, sem):
    wid = lax.axis_index("s") * NC + lax.axis_index("c")
    base = wid * B_PER_W

    pltpu.sync_copy(src_hbm.at[pl.ds(base, B_PER_W)], sidx_v)
    pltpu.sync_copy(dst_hbm.at[pl.ds(base, B_PER_W)], didx_v)

    copies = []
    for j in range(N_CHUNKS):
        rows = pl.ds(j * GATHER_CHUNK, GATHER_CHUNK)
        copies.append(pltpu.async_copy(
            in_hbm.at[sidx_v.at[rows]], a_v.at[rows], sem))
        copies.append(pltpu.async_copy(
            out_hbm.at[didx_v.at[rows]], b_v.at[rows], sem))
    for c in copies:
        c.wait()

    iota = lax.iota(jnp.int32, L)
    col_idx = [iota * L + k for k in range(L)]

    @pl.loop(0, GROUPS)
    def _group(g):
        row0 = g * L
        for r in range(L):
            row = row0 + r
            p0 = a_v[row, pl.ds(0, L)] * b_v[row, pl.ds(0, L)]
            p1 = a_v[row, pl.ds(L, L)] * b_v[row, pl.ds(L, L)]
            p2 = a_v[row, pl.ds(2 * L, L)] * b_v[row, pl.ds(2 * L, L)]
            p3 = a_v[row, pl.ds(3 * L, L)] * b_v[row, pl.ds(3 * L, L)]
            s_v[pl.ds(r * L, L)] = (p0 + p1) + (p2 + p3)
        res = plsc.load_gather(s_v, [col_idx[0]])
        for k in range(1, L):
            res = res + plsc.load_gather(s_v, [col_idx[k]])
        o_v[pl.ds(row0, L)] = res

    pltpu.sync_copy(o_v, o_hbm.at[pl.ds(base, B_PER_W)])


@jax.jit
def kernel(src_idx, dst_idx, in_emb, out_emb):
    mesh = plsc.VectorSubcoreMesh(
        core_axis_name="c", subcore_axis_name="s",
        num_cores=NC, num_subcores=NS)
    f = pl.kernel(
        _sc_body,
        out_type=jax.ShapeDtypeStruct((BATCH,), jnp.float32),
        mesh=mesh,
        compiler_params=pltpu.CompilerParams(
            needs_layout_passes=False, use_tc_tiling_on_sc=False),
        scratch_types=[
            pltpu.VMEM((B_PER_W,), jnp.int32),
            pltpu.VMEM((B_PER_W,), jnp.int32),
            pltpu.VMEM((B_PER_W, DIM), jnp.float32),
            pltpu.VMEM((B_PER_W, DIM), jnp.float32),
            pltpu.VMEM((L * L,), jnp.float32),
            pltpu.VMEM((B_PER_W,), jnp.float32),
            pltpu.SemaphoreType.DMA,
        ],
    )
    return f(src_idx, dst_idx, in_emb, out_emb)


# speedup vs baseline: 2.1700x; 2.1700x over previous
import jax
import jax.numpy as jnp
from jax import lax
from jax.experimental import pallas as pl
from jax.experimental.pallas import tpu as pltpu
from jax.experimental.pallas import tpu_sc as plsc

NUM_NODES = 1000000
DIM = 64
BATCH = 16384

NC = 2
NS = 16
L = 16
NW = NC * NS
B_PER_W = BATCH // NW
CH = 32
N_CHUNKS = B_PER_W // CH
N_IDX_VECS = B_PER_W // L


def _sc_body(src_hbm, dst_hbm, in_hbm, out_hbm, o_hbm,
             sidx_v, didx_v, a_t, b_t, s_v, o_v, sem):
    wid = lax.axis_index("s") * NC + lax.axis_index("c")
    base = wid * B_PER_W

    pltpu.sync_copy(src_hbm.at[pl.ds(base, B_PER_W)], sidx_v)
    pltpu.sync_copy(dst_hbm.at[pl.ds(base, B_PER_W)], didx_v)

    iota = lax.iota(jnp.int32, L)
    col_idx = [iota * L + k for k in range(L)]

    @pl.loop(0, N_CHUNKS)
    def _chunk(g):
        lk0 = g * CH
        svecs = [sidx_v[pl.ds(lk0 + v * L, L)] for v in range(CH // L)]
        dvecs = [didx_v[pl.ds(lk0 + v * L, L)] for v in range(CH // L)]
        stile = [lax.shift_right_logical(v, 3) for v in svecs]
        dtile = [lax.shift_right_logical(v, 3) for v in dvecs]
        ssub = [v & 7 for v in svecs]
        dsub = [v & 7 for v in dvecs]
        copies = []
        for j in range(CH):
            v, e = j // L, j % L
            copies.append(pltpu.async_copy(in_hbm.at[stile[v][e]], a_t.at[j], sem))
            copies.append(pltpu.async_copy(out_hbm.at[dtile[v][e]], b_t.at[j], sem))
        for c in copies:
            c.wait()
        for grp in range(CH // L):
            for r in range(L):
                j = grp * L + r
                rs = ssub[grp][r]
                rd = dsub[grp][r]
                p0 = a_t[j, rs, pl.ds(0, L)] * b_t[j, rd, pl.ds(0, L)]
                p1 = a_t[j, rs, pl.ds(L, L)] * b_t[j, rd, pl.ds(L, L)]
                p2 = a_t[j, rs, pl.ds(2 * L, L)] * b_t[j, rd, pl.ds(2 * L, L)]
                p3 = a_t[j, rs, pl.ds(3 * L, L)] * b_t[j, rd, pl.ds(3 * L, L)]
                s_v[pl.ds(r * L, L)] = (p0 + p1) + (p2 + p3)
            res = plsc.load_gather(s_v, [col_idx[0]])
            for k in range(1, L):
                res = res + plsc.load_gather(s_v, [col_idx[k]])
            o_v[pl.ds(lk0 + grp * L, L)] = res

    pltpu.sync_copy(o_v, o_hbm.at[pl.ds(base, B_PER_W)])


@jax.jit
def kernel(src_idx, dst_idx, in_emb, out_emb):
    mesh = plsc.VectorSubcoreMesh(
        core_axis_name="c", subcore_axis_name="s",
        num_cores=NC, num_subcores=NS)
    f = pl.kernel(
        _sc_body,
        out_type=jax.ShapeDtypeStruct((BATCH,), jnp.float32),
        mesh=mesh,
        compiler_params=pltpu.CompilerParams(needs_layout_passes=False),
        scratch_types=[
            pltpu.VMEM((B_PER_W,), jnp.int32),
            pltpu.VMEM((B_PER_W,), jnp.int32),
            pltpu.VMEM((CH, 8, DIM), jnp.float32),
            pltpu.VMEM((CH, 8, DIM), jnp.float32),
            pltpu.VMEM((L * L,), jnp.float32),
            pltpu.VMEM((B_PER_W,), jnp.float32),
            pltpu.SemaphoreType.DMA,
        ],
    )
    in3 = in_emb.reshape(NUM_NODES // 8, 8, DIM)
    out3 = out_emb.reshape(NUM_NODES // 8, 8, DIM)
    return f(src_idx, dst_idx, in3, out3)
